# baseline (speedup 1.0000x reference)
import functools

import jax
import jax.numpy as jnp
from jax import lax
from jax.experimental import pallas as pl
from jax.experimental.pallas import tpu as pltpu
from jax.experimental.pallas import tpu_sc as plsc

_N_MID = 16384
_N_TGT = 2048
_E0 = 524288
_E1 = 65536
_D = 128
_ACC_ROWS = 2176
_CNT_ROWS = 16
_NC = 2
_NS = 16
_NW = _NC * _NS
_CHUNK = 128


def _sc_aggregate(table, src, dst, zeros, edges):
    eblk = edges // _NW
    iters = eblk // 16
    cpl = (iters + _CHUNK - 1) // _CHUNK
    lreg = cpl * _CHUNK
    cap = 16 * lreg
    nloop = cap // _CHUNK
    rows_per = _ACC_ROWS // _NS
    mesh = plsc.VectorSubcoreMesh(core_axis_name="c", subcore_axis_name="s")

    @functools.partial(
        pl.kernel,
        out_type=(jax.ShapeDtypeStruct((_NC, _ACC_ROWS, _D), jnp.float32),
                  jax.ShapeDtypeStruct((_NC, _CNT_ROWS, _D), jnp.float32)),
        mesh=mesh,
        scratch_types=[
            pltpu.VMEM((eblk,), jnp.int32),
            pltpu.VMEM((eblk,), jnp.int32),
            pltpu.VMEM((cap,), jnp.int32),
            pltpu.VMEM((nloop, _CHUNK), jnp.int32),
            pltpu.VMEM((_CNT_ROWS, _D), jnp.float32),
            pltpu.VMEM((16,), jnp.int32),
            pltpu.VMEM((2, _CHUNK, _D), jnp.float32),
            pltpu.VMEM_SHARED((_ACC_ROWS, _D), jnp.float32),
            pltpu.VMEM_SHARED((_CNT_ROWS, _D), jnp.float32),
            pltpu.SemaphoreType.DMA,
        ],
        compiler_params=pltpu.CompilerParams(
            use_tc_tiling_on_sc=False, needs_layout_passes=False),
    )
    def agg(tab_hbm, src_hbm, dst_hbm, z_hbm, dummy_hbm, zsrc_hbm,
            out_hbm, cnt_hbm,
            src_v, dst_v, srcc_v, dstc_v, cnt_v, idx_v, rows_v,
            acc_sh, cnt_sh, sem):
        c = lax.axis_index("c")
        s = lax.axis_index("s")
        wid = s * _NC + c
        pltpu.sync_copy(z_hbm.at[pl.ds(s * rows_per, rows_per)],
                        acc_sh.at[pl.ds(s * rows_per, rows_per)])
        pltpu.sync_copy(z_hbm.at[pl.ds(0, _CNT_ROWS)], cnt_v)

        @pl.when(s == 0)
        def _():
            pltpu.sync_copy(z_hbm.at[pl.ds(0, _CNT_ROWS)], cnt_sh)

        pltpu.sync_copy(src_hbm.at[pl.ds(wid * eblk, eblk)], src_v)
        pltpu.sync_copy(dst_hbm.at[pl.ds(wid * eblk, eblk)], dst_v)
        pltpu.sync_copy(dummy_hbm, dstc_v)
        pltpu.sync_copy(zsrc_hbm, srcc_v)
        lanes = lax.iota(jnp.int32, 16)
        idx_v[...] = lanes
        base = lanes * lreg
        ones16 = jnp.ones((16,), jnp.float32)

        @plsc.parallel_loop(0, iters, 1, unroll=4,
                            carry=jnp.zeros((16,), jnp.int32))
        def cnts(i, cnts):
            d = dst_v[pl.ds(i * 16, 16)]
            sv = src_v[pl.ds(i * 16, 16)]
            m = d < _N_TGT
            pos = base + cnts
            plsc.store_scatter(srcc_v, [pos], sv, mask=m)
            plsc.store_scatter(dstc_v, [pos >> 7, pos & 127], d, mask=m)
            plsc.addupdate_scatter(cnt_v, [d >> 7, d & 127], ones16, mask=m)
            return cnts + m.astype(jnp.int32)

        plsc.subcore_barrier()
        pltpu.sync_copy(cnt_v, cnt_sh.at[idx_v], add=True)

        def start(j):
            off = jnp.minimum(j * _CHUNK, cap - _CHUNK)
            pltpu.make_async_copy(
                tab_hbm.at[srcc_v.at[pl.ds(off, _CHUNK)]],
                rows_v.at[j % 2], sem).start()

        c_l = [cnts[l] for l in range(16)]

        @pl.when(0 < c_l[0])
        def _():
            start(0)

        for l in range(16):
            c_next = c_l[l + 1] if l < 15 else jnp.int32(0)

            def body(k, carry, l=l, c_next=c_next):
                j = l * cpl + k
                live = k * _CHUNK < c_l[l]

                @pl.when(live)
                def _():
                    pltpu.make_async_copy(
                        tab_hbm.at[srcc_v.at[pl.ds(j * _CHUNK, _CHUNK)]],
                        rows_v.at[j % 2], sem).wait()

                live_next = jnp.where(k + 1 < cpl,
                                      (k + 1) * _CHUNK < c_l[l],
                                      0 < c_next)

                @pl.when(live_next)
                def _():
                    start(j + 1)

                @pl.when(live)
                def _():
                    pltpu.sync_copy(rows_v.at[j % 2],
                                    acc_sh.at[dstc_v.at[j]], add=True)
                return carry

            lax.fori_loop(0, cpl, body, 0)
        plsc.subcore_barrier()
        pltpu.sync_copy(acc_sh.at[pl.ds(s * rows_per, rows_per)],
                        out_hbm.at[c, pl.ds(s * rows_per, rows_per)])

        @pl.when(s == 0)
        def _():
            pltpu.sync_copy(cnt_sh, cnt_hbm.at[c])

    dummy = jnp.full((nloop, _CHUNK), _N_TGT, jnp.int32)
    zsrc = jnp.zeros((cap,), jnp.int32)
    return agg(table, src, dst, zeros, dummy, zsrc)


def _tc_dense0(feat, cnt, x_dst, w_l, b_l, w_r):
    def body(p_ref, c_ref, x_ref, wl_ref, bl_ref, wr_ref, o_ref):
        ssum = p_ref[0, :_N_TGT, :] + p_ref[1, :_N_TGT, :]
        cnt2 = c_ref[0] + c_ref[1]
        mean = ssum / jnp.maximum(cnt2, 1.0)
        h = jnp.dot(mean, wl_ref[...], preferred_element_type=jnp.float32)
        h = h + bl_ref[...]
        h = h + jnp.dot(x_ref[...], wr_ref[...],
                        preferred_element_type=jnp.float32)
        o_ref[...] = jnp.maximum(h, 0.0)

    return pl.pallas_call(
        body,
        out_shape=jax.ShapeDtypeStruct((_N_TGT, _D), jnp.float32),
    )(feat, cnt, x_dst, w_l, b_l, w_r)


def _tc_dense1(feat, cnt, h, w_l, b_l, w_r):
    def body(p_ref, c_ref, h_ref, wl_ref, bl_ref, wr_ref, o_ref):
        ssum = p_ref[0, :_N_TGT, :] + p_ref[1, :_N_TGT, :]
        cnt2 = c_ref[0] + c_ref[1]
        mean = ssum / jnp.maximum(cnt2, 1.0)
        out = jnp.dot(mean, wl_ref[...], preferred_element_type=jnp.float32)
        out = out + bl_ref[...]
        out = out + jnp.dot(h_ref[...], wr_ref[...],
                            preferred_element_type=jnp.float32)
        o_ref[...] = out

    return pl.pallas_call(
        body,
        out_shape=jax.ShapeDtypeStruct((_N_TGT, _D), jnp.float32),
    )(feat, cnt, h, w_l, b_l, w_r)


def kernel(x, edge_index_0, edge_index_1, num_dst0, num_dst1,
           W_l0, b_l0, W_r0, W_l1, b_l1, W_r1):
    del num_dst0, num_dst1
    zeros = jnp.zeros((_ACC_ROWS, _D), jnp.float32)
    feat0, cnt0 = _sc_aggregate(x, edge_index_0[0], edge_index_0[1],
                                zeros, _E0)
    h = _tc_dense0(feat0, cnt0.reshape(_NC, _N_TGT, 1), x[:_N_TGT],
                   W_l0, b_l0.reshape(1, _D), W_r0)
    feat1, cnt1 = _sc_aggregate(h, edge_index_1[0], edge_index_1[1],
                                zeros, _E1)
    return _tc_dense1(feat1, cnt1.reshape(_NC, _N_TGT, 1), h,
                      W_l1, b_l1.reshape(1, _D), W_r1)


# speedup vs baseline: 2.5043x; 2.5043x over previous
import functools

import jax
import jax.numpy as jnp
from jax import lax
from jax.experimental import pallas as pl
from jax.experimental.pallas import tpu as pltpu
from jax.experimental.pallas import tpu_sc as plsc

_N_MID = 16384
_N_TGT = 2048
_E0 = 524288
_E1 = 65536
_D = 128
_ACC_ROWS = 2176
_CNT_ROWS = 16
_NC = 2
_NS = 16
_NW = _NC * _NS
_CHUNK = 128


def _sc_aggregate(table, src, dst, zeros, edges):
    eblk = edges // _NW
    iters = eblk // 16
    cpl = (iters + _CHUNK - 1) // _CHUNK
    lreg = cpl * _CHUNK
    cap = 16 * lreg
    nloop = cap // _CHUNK
    rows_per = _ACC_ROWS // _NS
    mesh = plsc.VectorSubcoreMesh(core_axis_name="c", subcore_axis_name="s")

    @functools.partial(
        pl.kernel,
        out_type=(jax.ShapeDtypeStruct((_NC, _ACC_ROWS, _D), jnp.float32),
                  jax.ShapeDtypeStruct((_NC, _CNT_ROWS, _D), jnp.float32)),
        mesh=mesh,
        scratch_types=[
            pltpu.VMEM((eblk,), jnp.int32),
            pltpu.VMEM((eblk,), jnp.int32),
            pltpu.VMEM((cap,), jnp.int32),
            pltpu.VMEM((nloop, _CHUNK), jnp.int32),
            pltpu.VMEM((_CNT_ROWS, _D), jnp.float32),
            pltpu.VMEM((16,), jnp.int32),
            pltpu.VMEM((2, _CHUNK, _D), jnp.float32),
            pltpu.VMEM_SHARED((_ACC_ROWS, _D), jnp.float32),
            pltpu.VMEM_SHARED((_CNT_ROWS, _D), jnp.float32),
            pltpu.SemaphoreType.DMA,
        ],
        compiler_params=pltpu.CompilerParams(
            use_tc_tiling_on_sc=False, needs_layout_passes=False),
    )
    def agg(tab_hbm, src_hbm, dst_hbm, z_hbm, dummy_hbm, zsrc_hbm,
            out_hbm, cnt_hbm,
            src_v, dst_v, srcc_v, dstc_v, cnt_v, idx_v, rows_v,
            acc_sh, cnt_sh, sem):
        c = lax.axis_index("c")
        s = lax.axis_index("s")
        wid = s * _NC + c
        pltpu.sync_copy(z_hbm.at[pl.ds(s * rows_per, rows_per)],
                        acc_sh.at[pl.ds(s * rows_per, rows_per)])
        pltpu.sync_copy(z_hbm.at[pl.ds(0, _CNT_ROWS)], cnt_v)

        @pl.when(s == 0)
        def _():
            pltpu.sync_copy(z_hbm.at[pl.ds(0, _CNT_ROWS)], cnt_sh)

        pltpu.sync_copy(src_hbm.at[pl.ds(wid * eblk, eblk)], src_v)
        pltpu.sync_copy(dst_hbm.at[pl.ds(wid * eblk, eblk)], dst_v)
        pltpu.sync_copy(dummy_hbm, dstc_v)
        pltpu.sync_copy(zsrc_hbm, srcc_v)
        lanes = lax.iota(jnp.int32, 16)
        idx_v[...] = lanes
        base = lanes * lreg
        ones16 = jnp.ones((16,), jnp.float32)

        @plsc.parallel_loop(0, iters, 1, unroll=4,
                            carry=jnp.zeros((16,), jnp.int32))
        def cnts(i, cnts):
            d = dst_v[pl.ds(i * 16, 16)]
            sv = src_v[pl.ds(i * 16, 16)]
            m = d < _N_TGT
            pos = cnts * 16 + lanes
            plsc.store_scatter(srcc_v, [pos], sv, mask=m)
            plsc.store_scatter(dstc_v, [pos >> 7, pos & 127], d, mask=m)
            plsc.addupdate_scatter(cnt_v, [d >> 7, d & 127], ones16, mask=m)
            return cnts + m.astype(jnp.int32)

        n = jnp.max(cnts) * 16
        plsc.subcore_barrier()
        pltpu.sync_copy(cnt_v, cnt_sh.at[idx_v], add=True)

        @pl.when(n > 0)
        def _():
            pltpu.make_async_copy(
                tab_hbm.at[srcc_v.at[pl.ds(0, _CHUNK)]],
                rows_v.at[0], sem).start()

        def chunk(j, carry):
            @pl.when(j * _CHUNK < n)
            def _():
                pltpu.make_async_copy(
                    tab_hbm.at[srcc_v.at[pl.ds(j * _CHUNK, _CHUNK)]],
                    rows_v.at[j % 2], sem).wait()

                @pl.when((j + 1) * _CHUNK < n)
                def _():
                    pltpu.make_async_copy(
                        tab_hbm.at[srcc_v.at[
                            pl.ds(jnp.minimum((j + 1) * _CHUNK,
                                              cap - _CHUNK), _CHUNK)]],
                        rows_v.at[(j + 1) % 2], sem).start()

                pltpu.sync_copy(rows_v.at[j % 2],
                                acc_sh.at[dstc_v.at[j]], add=True)
            return carry

        lax.fori_loop(0, nloop, chunk, 0)
        plsc.subcore_barrier()
        pltpu.sync_copy(acc_sh.at[pl.ds(s * rows_per, rows_per)],
                        out_hbm.at[c, pl.ds(s * rows_per, rows_per)])

        @pl.when(s == 0)
        def _():
            pltpu.sync_copy(cnt_sh, cnt_hbm.at[c])

    dummy = jnp.full((nloop, _CHUNK), _N_TGT, jnp.int32)
    zsrc = jnp.zeros((cap,), jnp.int32)
    return agg(table, src, dst, zeros, dummy, zsrc)


def _tc_dense0(feat, cnt, x_dst, w_l, b_l, w_r):
    def body(p_ref, c_ref, x_ref, wl_ref, bl_ref, wr_ref, o_ref):
        ssum = p_ref[0, :_N_TGT, :] + p_ref[1, :_N_TGT, :]
        cnt2 = c_ref[0] + c_ref[1]
        mean = ssum / jnp.maximum(cnt2, 1.0)
        h = jnp.dot(mean, wl_ref[...], preferred_element_type=jnp.float32)
        h = h + bl_ref[...]
        h = h + jnp.dot(x_ref[...], wr_ref[...],
                        preferred_element_type=jnp.float32)
        o_ref[...] = jnp.maximum(h, 0.0)

    return pl.pallas_call(
        body,
        out_shape=jax.ShapeDtypeStruct((_N_TGT, _D), jnp.float32),
    )(feat, cnt, x_dst, w_l, b_l, w_r)


def _tc_dense1(feat, cnt, h, w_l, b_l, w_r):
    def body(p_ref, c_ref, h_ref, wl_ref, bl_ref, wr_ref, o_ref):
        ssum = p_ref[0, :_N_TGT, :] + p_ref[1, :_N_TGT, :]
        cnt2 = c_ref[0] + c_ref[1]
        mean = ssum / jnp.maximum(cnt2, 1.0)
        out = jnp.dot(mean, wl_ref[...], preferred_element_type=jnp.float32)
        out = out + bl_ref[...]
        out = out + jnp.dot(h_ref[...], wr_ref[...],
                            preferred_element_type=jnp.float32)
        o_ref[...] = out

    return pl.pallas_call(
        body,
        out_shape=jax.ShapeDtypeStruct((_N_TGT, _D), jnp.float32),
    )(feat, cnt, h, w_l, b_l, w_r)


def kernel(x, edge_index_0, edge_index_1, num_dst0, num_dst1,
           W_l0, b_l0, W_r0, W_l1, b_l1, W_r1):
    del num_dst0, num_dst1
    zeros = jnp.zeros((_ACC_ROWS, _D), jnp.float32)
    feat0, cnt0 = _sc_aggregate(x, edge_index_0[0], edge_index_0[1],
                                zeros, _E0)
    h = _tc_dense0(feat0, cnt0.reshape(_NC, _N_TGT, 1), x[:_N_TGT],
                   W_l0, b_l0.reshape(1, _D), W_r0)
    feat1, cnt1 = _sc_aggregate(h, edge_index_1[0], edge_index_1[1],
                                zeros, _E1)
    return _tc_dense1(feat1, cnt1.reshape(_NC, _N_TGT, 1), h,
                      W_l1, b_l1.reshape(1, _D), W_r1)


# speedup vs baseline: 7.4657x; 2.9811x over previous
import functools

import jax
import jax.numpy as jnp
from jax import lax
from jax.experimental import pallas as pl
from jax.experimental.pallas import tpu as pltpu
from jax.experimental.pallas import tpu_sc as plsc

_N_MID = 16384
_N_TGT = 2048
_E0 = 524288
_E1 = 65536
_D = 128
_ACC_ROWS = 2176
_CNT_ROWS = 16
_NC = 2
_NS = 16
_NW = _NC * _NS
_CHUNK = 128


def _sc_aggregate(table, src, dst, zeros, edges):
    eblk = edges // _NW
    iters = eblk // 16
    cap = eblk + _CHUNK
    rows_per = _ACC_ROWS // _NS
    nloop = cap // _CHUNK
    mesh = plsc.VectorSubcoreMesh(core_axis_name="c", subcore_axis_name="s")

    @functools.partial(
        pl.kernel,
        out_type=(jax.ShapeDtypeStruct((_NC, _ACC_ROWS, _D), jnp.float32),
                  jax.ShapeDtypeStruct((_NC, _CNT_ROWS, _D), jnp.float32)),
        mesh=mesh,
        scratch_types=[
            pltpu.VMEM((eblk,), jnp.int32),
            pltpu.VMEM((eblk,), jnp.int32),
            pltpu.VMEM((cap,), jnp.int32),
            pltpu.VMEM((nloop + 2, _CHUNK), jnp.int32),
            pltpu.VMEM((_CNT_ROWS, _D), jnp.float32),
            pltpu.VMEM((16,), jnp.int32),
            pltpu.VMEM((2, _CHUNK, _D), jnp.float32),
            pltpu.VMEM_SHARED((_ACC_ROWS, _D), jnp.float32),
            pltpu.VMEM_SHARED((_CNT_ROWS, _D), jnp.float32),
            pltpu.SemaphoreType.DMA,
        ],
        compiler_params=pltpu.CompilerParams(
            use_tc_tiling_on_sc=False, needs_layout_passes=False),
    )
    def agg(tab_hbm, src_hbm, dst_hbm, z_hbm, out_hbm, cnt_hbm,
            src_v, dst_v, srcc_v, dstc_v, cnt_v, idx_v, rows_v,
            acc_sh, cnt_sh, sem):
        c = lax.axis_index("c")
        s = lax.axis_index("s")
        wid = s * _NC + c
        pltpu.sync_copy(z_hbm.at[pl.ds(s * rows_per, rows_per)],
                        acc_sh.at[pl.ds(s * rows_per, rows_per)])
        pltpu.sync_copy(z_hbm.at[pl.ds(0, _CNT_ROWS)], cnt_v)

        @pl.when(s == 0)
        def _():
            pltpu.sync_copy(z_hbm.at[pl.ds(0, _CNT_ROWS)], cnt_sh)

        pltpu.sync_copy(src_hbm.at[pl.ds(wid * eblk, eblk)], src_v)
        pltpu.sync_copy(dst_hbm.at[pl.ds(wid * eblk, eblk)], dst_v)
        lanes = lax.iota(jnp.int32, 16)
        idx_v[...] = lanes
        ones16 = jnp.ones((16,), jnp.float32)

        @plsc.parallel_loop(0, iters, 1, unroll=4,
                            carry=jnp.zeros((16,), jnp.int32))
        def nvec(i, nvec):
            d = dst_v[pl.ds(i * 16, 16)]
            sv = src_v[pl.ds(i * 16, 16)]
            m = d < _N_TGT
            mi = m.astype(jnp.int32)
            pos = nvec + plsc.cumsum(mi) - mi
            plsc.store_scatter(srcc_v, [pos], sv, mask=m)
            plsc.store_scatter(dstc_v, [pos >> 7, pos & 127], d, mask=m)
            plsc.addupdate_scatter(cnt_v, [d >> 7, d & 127], ones16, mask=m)
            return nvec + plsc.all_reduce_population_count(m)

        n = jnp.max(nvec)
        for k in range(8):
            ppos = n + k * 16 + lanes
            plsc.store_scatter(srcc_v, [ppos], jnp.zeros((16,), jnp.int32))
            plsc.store_scatter(dstc_v, [ppos >> 7, ppos & 127],
                               jnp.full((16,), _N_TGT, jnp.int32))
        plsc.subcore_barrier()
        pltpu.sync_copy(cnt_v, cnt_sh.at[idx_v], add=True)

        @pl.when(n > 0)
        def _():
            pltpu.make_async_copy(
                tab_hbm.at[srcc_v.at[pl.ds(0, _CHUNK)]],
                rows_v.at[0], sem).start()

        def chunk(j, carry):
            @pl.when(j * _CHUNK < n)
            def _():
                pltpu.make_async_copy(
                    tab_hbm.at[srcc_v.at[pl.ds(j * _CHUNK, _CHUNK)]],
                    rows_v.at[j % 2], sem).wait()

                @pl.when((j + 1) * _CHUNK < n)
                def _():
                    pltpu.make_async_copy(
                        tab_hbm.at[srcc_v.at[pl.ds((j + 1) * _CHUNK, _CHUNK)]],
                        rows_v.at[(j + 1) % 2], sem).start()

                pltpu.sync_copy(rows_v.at[j % 2],
                                acc_sh.at[dstc_v.at[j]], add=True)
            return carry

        lax.fori_loop(0, nloop, chunk, 0)
        plsc.subcore_barrier()
        pltpu.sync_copy(acc_sh.at[pl.ds(s * rows_per, rows_per)],
                        out_hbm.at[c, pl.ds(s * rows_per, rows_per)])

        @pl.when(s == 0)
        def _():
            pltpu.sync_copy(cnt_sh, cnt_hbm.at[c])

    return agg(table, src, dst, zeros)


def _tc_dense0(feat, cnt, x_dst, w_l, b_l, w_r):
    def body(p_ref, c_ref, x_ref, wl_ref, bl_ref, wr_ref, o_ref):
        ssum = p_ref[0, :_N_TGT, :] + p_ref[1, :_N_TGT, :]
        cnt2 = c_ref[0] + c_ref[1]
        mean = ssum / jnp.maximum(cnt2, 1.0)
        h = jnp.dot(mean, wl_ref[...], preferred_element_type=jnp.float32)
        h = h + bl_ref[...]
        h = h + jnp.dot(x_ref[...], wr_ref[...],
                        preferred_element_type=jnp.float32)
        o_ref[...] = jnp.maximum(h, 0.0)

    return pl.pallas_call(
        body,
        out_shape=jax.ShapeDtypeStruct((_N_TGT, _D), jnp.float32),
    )(feat, cnt, x_dst, w_l, b_l, w_r)


def _tc_dense1(feat, cnt, h, w_l, b_l, w_r):
    def body(p_ref, c_ref, h_ref, wl_ref, bl_ref, wr_ref, o_ref):
        ssum = p_ref[0, :_N_TGT, :] + p_ref[1, :_N_TGT, :]
        cnt2 = c_ref[0] + c_ref[1]
        mean = ssum / jnp.maximum(cnt2, 1.0)
        out = jnp.dot(mean, wl_ref[...], preferred_element_type=jnp.float32)
        out = out + bl_ref[...]
        out = out + jnp.dot(h_ref[...], wr_ref[...],
                            preferred_element_type=jnp.float32)
        o_ref[...] = out

    return pl.pallas_call(
        body,
        out_shape=jax.ShapeDtypeStruct((_N_TGT, _D), jnp.float32),
    )(feat, cnt, h, w_l, b_l, w_r)


def kernel(x, edge_index_0, edge_index_1, num_dst0, num_dst1,
           W_l0, b_l0, W_r0, W_l1, b_l1, W_r1):
    del num_dst0, num_dst1
    zeros = jnp.zeros((_ACC_ROWS, _D), jnp.float32)
    feat0, cnt0 = _sc_aggregate(x, edge_index_0[0], edge_index_0[1],
                                zeros, _E0)
    h = _tc_dense0(feat0, cnt0.reshape(_NC, _N_TGT, 1), x[:_N_TGT],
                   W_l0, b_l0.reshape(1, _D), W_r0)
    feat1, cnt1 = _sc_aggregate(h, edge_index_1[0], edge_index_1[1],
                                zeros, _E1)
    return _tc_dense1(feat1, cnt1.reshape(_NC, _N_TGT, 1), h,
                      W_l1, b_l1.reshape(1, _D), W_r1)
